# baseline (speedup 1.0000x reference)
import functools

import jax
import jax.numpy as jnp
from jax import lax
from jax.experimental import pallas as pl
from jax.experimental.pallas import tpu as pltpu
from jax.experimental.pallas import tpu_sc as plsc

DIM = 64
LANES = 16
CH = 64

_GDN = lax.GatherDimensionNumbers(
    offset_dims=(), collapsed_slice_dims=(0,), start_index_map=(0,))


def _permute(v, idx):
    return lax.gather(v, idx[:, None], _GDN, (1,),
                      mode=lax.GatherScatterMode.PROMISE_IN_BOUNDS)


@functools.lru_cache(maxsize=None)
def _build(B, n_entities, n_relations, nc, ns):
    nw = nc * ns
    b_per_w = B // nw
    n_chunks = b_per_w // CH
    mesh = plsc.VectorSubcoreMesh(core_axis_name="c", subcore_axis_name="s")

    @functools.partial(
        pl.kernel,
        mesh=mesh,
        compiler_params=pltpu.CompilerParams(use_tc_tiling_on_sc=True),
        out_type=jax.ShapeDtypeStruct((B,), jnp.float32),
        scratch_types=[
            pltpu.VMEM((b_per_w,), jnp.int32),
            pltpu.VMEM((b_per_w,), jnp.int32),
            pltpu.VMEM((b_per_w,), jnp.int32),
            pltpu.VMEM((CH, DIM), jnp.float32),
            pltpu.VMEM((CH, DIM), jnp.float32),
            pltpu.VMEM((CH, DIM), jnp.float32),
            pltpu.VMEM((b_per_w,), jnp.float32),
            pltpu.SemaphoreType.DMA,
            pltpu.SemaphoreType.DMA,
        ],
    )
    def dist_mult(e_hbm, r_hbm, hi_hbm, ri_hbm, ti_hbm, out_hbm,
                  idx_h, idx_r, idx_t, h_rows, r_rows, t_rows, scores,
                  sem0, sem1):
        wid = lax.axis_index("s") * nc + lax.axis_index("c")
        base = wid * b_per_w

        pltpu.sync_copy(hi_hbm.at[pl.ds(base, b_per_w)], idx_h)
        pltpu.sync_copy(ri_hbm.at[pl.ds(base, b_per_w)], idx_r)
        pltpu.sync_copy(ti_hbm.at[pl.ds(base, b_per_w)], idx_t)

        lane = lax.broadcasted_iota(jnp.int32, (LANES,), 0)
        perms = [lane ^ k for k in (8, 4, 2, 1)]

        sems = [sem0, sem1]

        def step(c, carry):
            group_cps = []
            for g in range(CH // LANES):
                cps = []
                sem = sems[g % 2]
                hv = idx_h[pl.ds(c * CH + g * LANES, LANES)]
                rv = idx_r[pl.ds(c * CH + g * LANES, LANES)]
                tv = idx_t[pl.ds(c * CH + g * LANES, LANES)]
                qh = lax.shift_right_logical(hv, 3)
                qt = lax.shift_right_logical(tv, 3)
                sh = hv & 7
                st = tv & 7
                for k in range(LANES):
                    i = g * LANES + k
                    cps.append(pltpu.async_copy(
                        e_hbm.at[qh[k], pl.ds(sh[k], 1)],
                        h_rows.at[pl.ds(i, 1)], sem))
                    cps.append(pltpu.async_copy(r_hbm.at[pl.ds(rv[k], 1)],
                                                r_rows.at[pl.ds(i, 1)], sem))
                    cps.append(pltpu.async_copy(
                        e_hbm.at[qt[k], pl.ds(st[k], 1)],
                        t_rows.at[pl.ds(i, 1)], sem))
                group_cps.append(cps)

            for g in range(CH // LANES):
                for cp in group_cps[g]:
                    cp.wait()
                vec = jnp.zeros((LANES,), jnp.float32)
                for k in range(LANES):
                    i = g * LANES + k
                    acc = jnp.zeros((LANES,), jnp.float32)
                    for cb in range(DIM // LANES):
                        cs = pl.ds(cb * LANES, LANES)
                        acc = acc + h_rows[i, cs] * r_rows[i, cs] * t_rows[i, cs]
                    for p in perms:
                        acc = acc + _permute(acc, p)
                    vec = jnp.where(lane == k, acc, vec)
                scores[pl.ds(c * CH + g * LANES, LANES)] = vec
            return carry

        lax.fori_loop(0, n_chunks, step, 0)

        pltpu.sync_copy(scores, out_hbm.at[pl.ds(base, b_per_w)])

    return dist_mult


def kernel(h_idx, r_idx, t_idx, E, R):
    B = h_idx.shape[0]
    info = plsc.get_sparse_core_info()
    f = _build(B, E.shape[0], R.shape[0], info.num_cores, info.num_subcores)
    E3 = E.reshape(E.shape[0] // 8, 8, DIM)
    return f(E3, R, h_idx.astype(jnp.int32), r_idx.astype(jnp.int32),
             t_idx.astype(jnp.int32))


# speedup vs baseline: 1.0482x; 1.0482x over previous
import functools

import jax
import jax.numpy as jnp
from jax import lax
from jax.experimental import pallas as pl
from jax.experimental.pallas import tpu as pltpu
from jax.experimental.pallas import tpu_sc as plsc

DIM = 64
LANES = 16
CH = 32

_GDN = lax.GatherDimensionNumbers(
    offset_dims=(), collapsed_slice_dims=(0,), start_index_map=(0,))


def _permute(v, idx):
    return lax.gather(v, idx[:, None], _GDN, (1,),
                      mode=lax.GatherScatterMode.PROMISE_IN_BOUNDS)


@functools.lru_cache(maxsize=None)
def _build(B, n_entities, n_relations, nc, ns):
    nw = nc * ns
    b_per_w = B // nw
    n_chunks = b_per_w // CH
    mesh = plsc.VectorSubcoreMesh(core_axis_name="c", subcore_axis_name="s")

    @functools.partial(
        pl.kernel,
        mesh=mesh,
        compiler_params=pltpu.CompilerParams(use_tc_tiling_on_sc=True),
        out_type=jax.ShapeDtypeStruct((B,), jnp.float32),
        scratch_types=[
            pltpu.VMEM((b_per_w,), jnp.int32),
            pltpu.VMEM((b_per_w,), jnp.int32),
            pltpu.VMEM((b_per_w,), jnp.int32),
            pltpu.VMEM((CH, DIM), jnp.float32),
            pltpu.VMEM((CH, DIM), jnp.float32),
            pltpu.VMEM((CH, DIM), jnp.float32),
            pltpu.VMEM((b_per_w,), jnp.float32),
            pltpu.SemaphoreType.DMA,
            pltpu.SemaphoreType.DMA,
        ],
    )
    def dist_mult(e_hbm, r_hbm, hi_hbm, ri_hbm, ti_hbm, out_hbm,
                  idx_h, idx_r, idx_t, h_rows, r_rows, t_rows, scores,
                  sem0, sem1):
        wid = lax.axis_index("s") * nc + lax.axis_index("c")
        base = wid * b_per_w

        pltpu.sync_copy(hi_hbm.at[pl.ds(base, b_per_w)], idx_h)
        pltpu.sync_copy(ri_hbm.at[pl.ds(base, b_per_w)], idx_r)
        pltpu.sync_copy(ti_hbm.at[pl.ds(base, b_per_w)], idx_t)

        lane = lax.broadcasted_iota(jnp.int32, (LANES,), 0)
        perms = [lane ^ k for k in (8, 4, 2, 1)]

        sems = [sem0, sem1]

        def step(c, carry):
            group_cps = []
            for g in range(CH // LANES):
                cps = []
                sem = sems[g % 2]
                hv = idx_h[pl.ds(c * CH + g * LANES, LANES)]
                rv = idx_r[pl.ds(c * CH + g * LANES, LANES)]
                tv = idx_t[pl.ds(c * CH + g * LANES, LANES)]
                qh = lax.shift_right_logical(hv, 3)
                qt = lax.shift_right_logical(tv, 3)
                sh = hv & 7
                st = tv & 7
                for k in range(LANES):
                    i = g * LANES + k
                    cps.append(pltpu.async_copy(
                        e_hbm.at[qh[k], pl.ds(sh[k], 1)],
                        h_rows.at[pl.ds(i, 1)], sem))
                    cps.append(pltpu.async_copy(r_hbm.at[pl.ds(rv[k], 1)],
                                                r_rows.at[pl.ds(i, 1)], sem))
                    cps.append(pltpu.async_copy(
                        e_hbm.at[qt[k], pl.ds(st[k], 1)],
                        t_rows.at[pl.ds(i, 1)], sem))
                group_cps.append(cps)

            for g in range(CH // LANES):
                for cp in group_cps[g]:
                    cp.wait()
                vec = jnp.zeros((LANES,), jnp.float32)
                for k in range(LANES):
                    i = g * LANES + k
                    acc = jnp.zeros((LANES,), jnp.float32)
                    for cb in range(DIM // LANES):
                        cs = pl.ds(cb * LANES, LANES)
                        acc = acc + h_rows[i, cs] * r_rows[i, cs] * t_rows[i, cs]
                    for p in perms:
                        acc = acc + _permute(acc, p)
                    vec = jnp.where(lane == k, acc, vec)
                scores[pl.ds(c * CH + g * LANES, LANES)] = vec
            return carry

        lax.fori_loop(0, n_chunks, step, 0)

        pltpu.sync_copy(scores, out_hbm.at[pl.ds(base, b_per_w)])

    return dist_mult


def kernel(h_idx, r_idx, t_idx, E, R):
    B = h_idx.shape[0]
    info = plsc.get_sparse_core_info()
    f = _build(B, E.shape[0], R.shape[0], info.num_cores, info.num_subcores)
    E3 = E.reshape(E.shape[0] // 8, 8, DIM)
    return f(E3, R, h_idx.astype(jnp.int32), r_idx.astype(jnp.int32),
             t_idx.astype(jnp.int32))
